# baseline (speedup 1.0000x reference)
import functools

import jax
import jax.numpy as jnp
from jax import lax
from jax.experimental import pallas as pl
from jax.experimental.pallas import tpu as pltpu
from jax.experimental.pallas import tpu_sc as plsc

N_NODES = 10000
N_EDGES = 160000
D_FEAT = 128
UNITS = 2
EPS = 1e-07

NCHUNK = 8
CF = 16
NCH = 8
ROW = NCH * CF

NSUB = 16
NCORE = 2
EDGES_PER_SUB = N_EDGES // NSUB
EPB = 80
NBLK = EDGES_PER_SUB // EPB
NPAD = 10240
NODES_PER_SUB = NPAD // NSUB
ZROWS = 64

EB1 = 2000
NB3 = 1280


def _stage1_body(x_ref, o_ref):
    x = jnp.clip(x_ref[...], EPS, 1.0 - EPS)
    l1 = jnp.log(x)
    l2 = jnp.log(1.0 - x)
    for c in range(NCHUNK):
        o_ref[c, :, 0:CF] = l1[:, c * CF:(c + 1) * CF]
        o_ref[c, :, CF:2 * CF] = l2[:, c * CF:(c + 1) * CF]


def _stage1(data):
    return pl.pallas_call(
        _stage1_body,
        grid=(N_EDGES // EB1,),
        in_specs=[pl.BlockSpec((EB1, D_FEAT), lambda i: (i, 0))],
        out_specs=pl.BlockSpec((NCHUNK, EB1, 2 * CF), lambda i: (0, i, 0)),
        out_shape=jax.ShapeDtypeStruct((NCHUNK, N_EDGES, 2 * CF), jnp.float32),
    )(data)


ZCOPY = 8


def _sc_body(l_hbm, idx_hbm, p_hbm, out_hbm,
             l_buf, idx3, rows2, p_buf, acc,
             sem_in, sem_out):
    cid = lax.axis_index("c")
    sid = lax.axis_index("s")

    pltpu.sync_copy(p_hbm, p_buf)
    pv = [p_buf[ch] for ch in range(NCH)]

    z16 = jnp.zeros((16,), jnp.float32)

    for r in range(NCHUNK // NCORE):
        chunk = r * NCORE + cid

        def in_l(blk, par):
            base = sid * EDGES_PER_SUB + blk * EPB
            return pltpu.make_async_copy(
                l_hbm.at[chunk, pl.ds(base, EPB)], l_buf.at[par], sem_in)

        def in_i(blk, par3):
            return pltpu.make_async_copy(
                idx_hbm.at[sid, blk], idx3.at[par3], sem_in)

        def zfill(i, carry):
            for t in range(ROW // 16):
                rows2[0, i, pl.ds(t * 16, 16)] = z16
            return carry

        lax.fori_loop(0, EPB, zfill, 0)
        zd = [pltpu.async_copy(
                  rows2.at[0],
                  acc.at[pl.ds(sid * NODES_PER_SUB + z * EPB, EPB)],
                  sem_in)
              for z in range(ZCOPY)]
        for d in zd:
            d.wait()
        plsc.subcore_barrier()

        in_l(0, 0).start()
        in_i(0, 0).start()

        def blk_body(k, carry):
            p2 = lax.rem(k, 2)
            p3 = lax.rem(k, 3)
            in_l(k, p2).wait()
            in_i(k, p3).wait()

            @pl.when(k < NBLK - 1)
            def _():
                in_l(k + 1, 1 - p2).start()
                in_i(k + 1, lax.rem(k + 1, 3)).start()

            @plsc.parallel_loop(0, EPB, unroll=4)
            def edge(j):
                l1 = l_buf[p2, j, 0:CF]
                l2 = l_buf[p2, j, CF:2 * CF]
                for ch in range(NCH):
                    lv = l1 if ((ch // 2) % 2 == 0) else l2
                    rows2[p2, j, pl.ds(ch * CF, CF)] = jnp.exp(pv[ch] * lv)

            @pl.when(k > 0)
            def _():
                pltpu.make_async_copy(
                    rows2.at[1 - p2], acc.at[idx3.at[lax.rem(k + 2, 3)]],
                    sem_out).wait()

            pltpu.async_copy(
                rows2.at[p2], acc.at[idx3.at[p3]], sem_out, add=True)
            return carry

        lax.fori_loop(0, NBLK, blk_body, 0)
        pltpu.make_async_copy(
            rows2.at[(NBLK - 1) % 2], acc.at[idx3.at[(NBLK - 1) % 3]],
            sem_out).wait()
        plsc.subcore_barrier()

        pltpu.sync_copy(
            acc.at[pl.ds(sid * NODES_PER_SUB, NODES_PER_SUB)],
            out_hbm.at[chunk, pl.ds(sid * NODES_PER_SUB, NODES_PER_SUB)])
        if r != NCHUNK // NCORE - 1:
            plsc.subcore_barrier()


def _sc_scatter(l_arr, index, p_arr):
    mesh = plsc.VectorSubcoreMesh(core_axis_name="c", subcore_axis_name="s")
    f = pl.kernel(
        _sc_body,
        out_type=jax.ShapeDtypeStruct((NCHUNK, NPAD, ROW), jnp.float32),
        mesh=mesh,
        scratch_types=[
            pltpu.VMEM((2, EPB, 2 * CF), jnp.float32),
            pltpu.VMEM((3, EPB), jnp.int32),
            pltpu.VMEM((2, EPB, ROW), jnp.float32),
            pltpu.VMEM((NCH, 16), jnp.float32),
            pltpu.VMEM_SHARED((NPAD, ROW), jnp.float32),
            pltpu.SemaphoreType.DMA,
            pltpu.SemaphoreType.DMA,
        ],
    )
    return f(l_arr, index.reshape(NSUB, NBLK, EPB), p_arr)


def _stage3_body(s_ref, qb_ref, o_ref):
    for c in range(NCHUNK):
        terms = []
        for ch in range(NCH):
            s = jnp.clip(s_ref[c, :, ch * CF:(ch + 1) * CF], EPS, None)
            q = qb_ref[0, ch:ch + 1, 0:CF]
            ab = qb_ref[1, ch:ch + 1, 0:CF]
            terms.append(jnp.exp(q * jnp.log(s)) * ab)
        for u in range(UNITS):
            num = terms[u] + terms[2 + u]
            den = terms[4 + u] + terms[6 + u]
            mult = 2.0 * jnp.clip(jnp.sign(den), 0.0, None) - 1.0
            den = jnp.where((den < EPS) & (den > -EPS), mult * EPS, den)
            o_ref[u, :, c * CF:(c + 1) * CF] = num / den


def _stage3(s_arr, qb_arr):
    return pl.pallas_call(
        _stage3_body,
        grid=(NPAD // NB3,),
        in_specs=[
            pl.BlockSpec((NCHUNK, NB3, ROW), lambda nb: (0, nb, 0)),
            pl.BlockSpec((2, NCH, 128), lambda nb: (0, 0, 0)),
        ],
        out_specs=pl.BlockSpec((UNITS, NB3, D_FEAT), lambda nb: (0, nb, 0)),
        out_shape=jax.ShapeDtypeStruct((UNITS, NPAD, D_FEAT), jnp.float32),
    )(s_arr, qb_arr)


def kernel(data, index, W):
    p = jax.nn.relu(W[0:4]).reshape(NCH)
    p_arr = jnp.broadcast_to(p[:, None], (NCH, 16))
    q = jax.nn.relu(W[4:8]).reshape(NCH)
    ab = W[8:12].reshape(NCH)
    qb_arr = jnp.stack([
        jnp.broadcast_to(q[:, None], (NCH, 128)),
        jnp.broadcast_to(ab[:, None], (NCH, 128)),
    ])
    l_arr = _stage1(data)
    s_arr = _sc_scatter(l_arr, index, p_arr)
    r_arr = _stage3(s_arr, qb_arr)
    return jnp.transpose(r_arr, (1, 2, 0))[:N_NODES]


# speedup vs baseline: 1.0030x; 1.0030x over previous
import functools

import jax
import jax.numpy as jnp
from jax import lax
from jax.experimental import pallas as pl
from jax.experimental.pallas import tpu as pltpu
from jax.experimental.pallas import tpu_sc as plsc

N_NODES = 10000
N_EDGES = 160000
D_FEAT = 128
UNITS = 2
EPS = 1e-07

NCHUNK = 8
CF = 16
NCH = 8
ROW = NCH * CF

NSUB = 16
NCORE = 2
EDGES_PER_SUB = N_EDGES // NSUB
EPB = 80
NBLK = EDGES_PER_SUB // EPB
NPAD = 10240
NODES_PER_SUB = NPAD // NSUB
ZROWS = 64

EB1 = 2000
NB3 = 1280


def _stage1_body(x_ref, o_ref):
    x = jnp.clip(x_ref[...], EPS, 1.0 - EPS)
    l1 = jnp.log(x)
    l2 = jnp.log(1.0 - x)
    for c in range(NCHUNK):
        o_ref[c, :, 0:CF] = l1[:, c * CF:(c + 1) * CF]
        o_ref[c, :, CF:2 * CF] = l2[:, c * CF:(c + 1) * CF]


def _stage1(data):
    return pl.pallas_call(
        _stage1_body,
        grid=(N_EDGES // EB1,),
        in_specs=[pl.BlockSpec((EB1, D_FEAT), lambda i: (i, 0))],
        out_specs=pl.BlockSpec((NCHUNK, EB1, 2 * CF), lambda i: (0, i, 0)),
        out_shape=jax.ShapeDtypeStruct((NCHUNK, N_EDGES, 2 * CF), jnp.float32),
    )(data)


ZCOPY = 8


def _sc_body(l_hbm, idx_hbm, p_hbm, out_hbm,
             l_buf, idx3, rows2, p_buf, acc,
             sem_in, sem_out):
    cid = lax.axis_index("c")
    sid = lax.axis_index("s")

    pltpu.sync_copy(p_hbm, p_buf)
    pv = [p_buf[ch] for ch in range(NCH)]

    z16 = jnp.zeros((16,), jnp.float32)

    def round_body(r, rcarry):
        chunk = r * NCORE + cid

        def in_l(blk, par):
            base = sid * EDGES_PER_SUB + blk * EPB
            return pltpu.make_async_copy(
                l_hbm.at[chunk, pl.ds(base, EPB)], l_buf.at[par], sem_in)

        def in_i(blk, par3):
            return pltpu.make_async_copy(
                idx_hbm.at[sid, blk], idx3.at[par3], sem_in)

        def zfill(i, carry):
            for t in range(ROW // 16):
                rows2[0, i, pl.ds(t * 16, 16)] = z16
            return carry

        lax.fori_loop(0, EPB, zfill, 0)
        zd = [pltpu.async_copy(
                  rows2.at[0],
                  acc.at[pl.ds(sid * NODES_PER_SUB + z * EPB, EPB)],
                  sem_in)
              for z in range(ZCOPY)]
        for d in zd:
            d.wait()
        plsc.subcore_barrier()

        in_l(0, 0).start()
        in_i(0, 0).start()

        def blk_body(k, carry):
            p2 = lax.rem(k, 2)
            p3 = lax.rem(k, 3)
            in_l(k, p2).wait()
            in_i(k, p3).wait()

            @pl.when(k < NBLK - 1)
            def _():
                in_l(k + 1, 1 - p2).start()
                in_i(k + 1, lax.rem(k + 1, 3)).start()

            for j in range(EPB):
                l1 = l_buf[p2, j, 0:CF]
                l2 = l_buf[p2, j, CF:2 * CF]
                for ch in range(NCH):
                    lv = l1 if ((ch // 2) % 2 == 0) else l2
                    rows2[p2, j, pl.ds(ch * CF, CF)] = jnp.exp(pv[ch] * lv)

            @pl.when(k > 0)
            def _():
                pltpu.make_async_copy(
                    rows2.at[1 - p2], acc.at[idx3.at[lax.rem(k + 2, 3)]],
                    sem_out).wait()

            pltpu.async_copy(
                rows2.at[p2], acc.at[idx3.at[p3]], sem_out, add=True)
            return carry

        lax.fori_loop(0, NBLK, blk_body, 0)
        pltpu.make_async_copy(
            rows2.at[(NBLK - 1) % 2], acc.at[idx3.at[(NBLK - 1) % 3]],
            sem_out).wait()
        plsc.subcore_barrier()

        pltpu.sync_copy(
            acc.at[pl.ds(sid * NODES_PER_SUB, NODES_PER_SUB)],
            out_hbm.at[chunk, pl.ds(sid * NODES_PER_SUB, NODES_PER_SUB)])
        plsc.subcore_barrier()
        return rcarry

    lax.fori_loop(0, NCHUNK // NCORE, round_body, 0)


def _sc_scatter(l_arr, index, p_arr):
    mesh = plsc.VectorSubcoreMesh(core_axis_name="c", subcore_axis_name="s")
    f = pl.kernel(
        _sc_body,
        out_type=jax.ShapeDtypeStruct((NCHUNK, NPAD, ROW), jnp.float32),
        mesh=mesh,
        scratch_types=[
            pltpu.VMEM((2, EPB, 2 * CF), jnp.float32),
            pltpu.VMEM((3, EPB), jnp.int32),
            pltpu.VMEM((2, EPB, ROW), jnp.float32),
            pltpu.VMEM((NCH, 16), jnp.float32),
            pltpu.VMEM_SHARED((NPAD, ROW), jnp.float32),
            pltpu.SemaphoreType.DMA,
            pltpu.SemaphoreType.DMA,
        ],
    )
    return f(l_arr, index.reshape(NSUB, NBLK, EPB), p_arr)


def _stage3_body(s_ref, qb_ref, o_ref):
    for c in range(NCHUNK):
        terms = []
        for ch in range(NCH):
            s = jnp.clip(s_ref[c, :, ch * CF:(ch + 1) * CF], EPS, None)
            q = qb_ref[0, ch:ch + 1, 0:CF]
            ab = qb_ref[1, ch:ch + 1, 0:CF]
            terms.append(jnp.exp(q * jnp.log(s)) * ab)
        for u in range(UNITS):
            num = terms[u] + terms[2 + u]
            den = terms[4 + u] + terms[6 + u]
            mult = 2.0 * jnp.clip(jnp.sign(den), 0.0, None) - 1.0
            den = jnp.where((den < EPS) & (den > -EPS), mult * EPS, den)
            o_ref[u, :, c * CF:(c + 1) * CF] = num / den


def _stage3(s_arr, qb_arr):
    return pl.pallas_call(
        _stage3_body,
        grid=(NPAD // NB3,),
        in_specs=[
            pl.BlockSpec((NCHUNK, NB3, ROW), lambda nb: (0, nb, 0)),
            pl.BlockSpec((2, NCH, 128), lambda nb: (0, 0, 0)),
        ],
        out_specs=pl.BlockSpec((UNITS, NB3, D_FEAT), lambda nb: (0, nb, 0)),
        out_shape=jax.ShapeDtypeStruct((UNITS, NPAD, D_FEAT), jnp.float32),
    )(s_arr, qb_arr)


def kernel(data, index, W):
    p = jax.nn.relu(W[0:4]).reshape(NCH)
    p_arr = jnp.broadcast_to(p[:, None], (NCH, 16))
    q = jax.nn.relu(W[4:8]).reshape(NCH)
    ab = W[8:12].reshape(NCH)
    qb_arr = jnp.stack([
        jnp.broadcast_to(q[:, None], (NCH, 128)),
        jnp.broadcast_to(ab[:, None], (NCH, 128)),
    ])
    l_arr = _stage1(data)
    s_arr = _sc_scatter(l_arr, index, p_arr)
    r_arr = _stage3(s_arr, qb_arr)
    return jnp.transpose(r_arr, (1, 2, 0))[:N_NODES]


# speedup vs baseline: 1.5206x; 1.5161x over previous
import jax
import jax.numpy as jnp
from jax import lax
from jax.experimental import pallas as pl
from jax.experimental.pallas import tpu as pltpu
from jax.experimental.pallas import tpu_sc as plsc

N_NODES = 10000
N_EDGES = 160000
D_FEAT = 128
UNITS = 2
EPS = 1e-07

NCH = 8
NSUB = 16
NCORE = 2
EDGES_PER_SUB = N_EDGES // NSUB
EPB = 80
NBLK = EDGES_PER_SUB // EPB
NPAD = 10240
NODES_PER_SUB = NPAD // NSUB
NSLOT = 3
ZCOPY = NODES_PER_SUB // EPB

EB1 = 2000
NB3 = 1280


def _stage1_body(x_ref, o_ref):
    x = jnp.clip(x_ref[...], EPS, 1.0 - EPS)
    o_ref[0] = jnp.log(x)
    o_ref[1] = jnp.log(1.0 - x)


def _stage1(data):
    return pl.pallas_call(
        _stage1_body,
        grid=(N_EDGES // EB1,),
        in_specs=[pl.BlockSpec((EB1, D_FEAT), lambda i: (i, 0))],
        out_specs=pl.BlockSpec((2, EB1, D_FEAT), lambda i: (0, i, 0)),
        out_shape=jax.ShapeDtypeStruct((2, N_EDGES, D_FEAT), jnp.float32),
    )(data)


def _sc_body(l_hbm, idx_hbm, p_hbm, out_hbm,
             buf, idx_all, p_buf, acc, sem_a, sem_b, sem_out):
    cid = lax.axis_index("c")
    sid = lax.axis_index("s")

    pltpu.sync_copy(p_hbm, p_buf)
    pltpu.sync_copy(idx_hbm.at[sid], idx_all)

    z16 = jnp.zeros((16,), jnp.float32)

    def round_body(r, rcarry):
        ch = r * NCORE + cid
        b01 = lax.rem(r, 2)
        pch = p_buf[pl.ds(ch * 16, 16)]

        def in_cp(blk, sem):
            base = sid * EDGES_PER_SUB + blk * EPB
            slot = lax.rem(blk, NSLOT)
            return pltpu.make_async_copy(
                l_hbm.at[pl.ds(b01 * N_EDGES + base, EPB)],
                buf.at[slot], sem)

        def scat_cp(blk):
            return pltpu.make_async_copy(
                buf.at[lax.rem(blk, NSLOT)],
                acc.at[idx_all.at[blk]], sem_out)

        def zfill(i, carry):
            for t in range(D_FEAT // 16):
                buf[0, i, pl.ds(t * 16, 16)] = z16
            return carry

        lax.fori_loop(0, EPB, zfill, 0)
        zd = [pltpu.async_copy(
                  buf.at[0],
                  acc.at[pl.ds(sid * NODES_PER_SUB + z * EPB, EPB)],
                  sem_a)
              for z in range(ZCOPY)]
        for d in zd:
            d.wait()
        plsc.subcore_barrier()

        in_cp(0, sem_a).start()
        in_cp(1, sem_b).start()

        def do_block(k, sem, first):
            slot = lax.rem(k, NSLOT)
            in_cp(k, sem).wait()

            @plsc.parallel_loop(0, EPB, unroll=8)
            def edge(j):
                for t in range(NCH):
                    lv = buf[slot, j, pl.ds(t * 16, 16)]
                    buf[slot, j, pl.ds(t * 16, 16)] = jnp.exp(pch * lv)

            if first:
                @pl.when(k > 0)
                def _():
                    scat_cp(k - 1).wait()
            else:
                scat_cp(k - 1).wait()

            @pl.when(k + 2 < NBLK)
            def _():
                in_cp(k + 2, sem).start()

            pltpu.async_copy(
                buf.at[slot], acc.at[idx_all.at[k]], sem_out, add=True)

        def pair_body(m, carry):
            do_block(2 * m, sem_a, True)
            do_block(2 * m + 1, sem_b, False)
            return carry

        lax.fori_loop(0, (NBLK - 1) // 2, pair_body, 0)
        do_block(NBLK - 1, sem_a, False)
        scat_cp(NBLK - 1).wait()
        plsc.subcore_barrier()

        pltpu.sync_copy(
            acc.at[pl.ds(sid * NODES_PER_SUB, NODES_PER_SUB)],
            out_hbm.at[pl.ds(ch * NPAD + sid * NODES_PER_SUB, NODES_PER_SUB)])
        plsc.subcore_barrier()
        return rcarry

    lax.fori_loop(0, NCH // NCORE, round_body, 0)


def _sc_scatter(l_arr, index, p_arr):
    mesh = plsc.VectorSubcoreMesh(core_axis_name="c", subcore_axis_name="s")
    f = pl.kernel(
        _sc_body,
        out_type=jax.ShapeDtypeStruct((NCH * NPAD, D_FEAT), jnp.float32),
        mesh=mesh,
        scratch_types=[
            pltpu.VMEM((NSLOT, EPB, D_FEAT), jnp.float32),
            pltpu.VMEM((NBLK, EPB), jnp.int32),
            pltpu.VMEM((NCH * 16,), jnp.float32),
            pltpu.VMEM_SHARED((NPAD, D_FEAT), jnp.float32),
            pltpu.SemaphoreType.DMA,
            pltpu.SemaphoreType.DMA,
            pltpu.SemaphoreType.DMA,
        ],
    )
    return f(l_arr.reshape(2 * N_EDGES, D_FEAT),
             index.reshape(NSUB, NBLK, EPB), p_arr)


def _stage3_body(s_ref, qb_ref, o_ref):
    terms = []
    for ch in range(NCH):
        s = jnp.clip(s_ref[ch], EPS, None)
        q = qb_ref[0, ch:ch + 1, :]
        ab = qb_ref[1, ch:ch + 1, :]
        terms.append(jnp.exp(q * jnp.log(s)) * ab)
    for u in range(UNITS):
        num = terms[u] + terms[2 + u]
        den = terms[4 + u] + terms[6 + u]
        mult = 2.0 * jnp.clip(jnp.sign(den), 0.0, None) - 1.0
        den = jnp.where((den < EPS) & (den > -EPS), mult * EPS, den)
        o_ref[u] = num / den


def _stage3(s_arr, qb_arr):
    return pl.pallas_call(
        _stage3_body,
        grid=(NPAD // NB3,),
        in_specs=[
            pl.BlockSpec((NCH, NB3, D_FEAT), lambda nb: (0, nb, 0)),
            pl.BlockSpec((2, NCH, 128), lambda nb: (0, 0, 0)),
        ],
        out_specs=pl.BlockSpec((UNITS, NB3, D_FEAT), lambda nb: (0, nb, 0)),
        out_shape=jax.ShapeDtypeStruct((UNITS, NPAD, D_FEAT), jnp.float32),
    )(s_arr, qb_arr)


def kernel(data, index, W):
    p = jax.nn.relu(W[0:4]).reshape(NCH)
    p_arr = jnp.broadcast_to(p[:, None], (NCH, 16)).reshape(NCH * 16)
    q = jax.nn.relu(W[4:8]).reshape(NCH)
    ab = W[8:12].reshape(NCH)
    qb_arr = jnp.stack([
        jnp.broadcast_to(q[:, None], (NCH, 128)),
        jnp.broadcast_to(ab[:, None], (NCH, 128)),
    ])
    l_arr = _stage1(data)
    s_arr = _sc_scatter(l_arr, index, p_arr).reshape(NCH, NPAD, D_FEAT)
    r_arr = _stage3(s_arr, qb_arr)
    return jnp.transpose(r_arr, (1, 2, 0))[:N_NODES]


# speedup vs baseline: 1.5921x; 1.0470x over previous
import jax
import jax.numpy as jnp
from jax import lax
from jax.experimental import pallas as pl
from jax.experimental.pallas import tpu as pltpu
from jax.experimental.pallas import tpu_sc as plsc

N_NODES = 10000
N_EDGES = 160000
D_FEAT = 128
UNITS = 2
EPS = 1e-07

NCH = 8
NSUB = 16
NCORE = 2
EDGES_PER_SUB = N_EDGES // NSUB
EPB = 80
NBLK = EDGES_PER_SUB // EPB
NPAD = 10240
NODES_PER_SUB = NPAD // NSUB
NSLOT = 3
ZCOPY = NODES_PER_SUB // EPB

EB1 = 2000
NB3 = 1280


def _stage1_body_a(x_ref, o_ref):
    o_ref[...] = jnp.log(jnp.clip(x_ref[...], EPS, 1.0 - EPS))


def _stage1_body_b(x_ref, o_ref):
    o_ref[...] = jnp.log(1.0 - jnp.clip(x_ref[...], EPS, 1.0 - EPS))


def _stage1(data, which):
    return pl.pallas_call(
        _stage1_body_a if which == 0 else _stage1_body_b,
        grid=(N_EDGES // EB1,),
        in_specs=[pl.BlockSpec((EB1, D_FEAT), lambda i: (i, 0))],
        out_specs=pl.BlockSpec((EB1, D_FEAT), lambda i: (i, 0)),
        out_shape=jax.ShapeDtypeStruct((N_EDGES, D_FEAT), jnp.float32),
    )(data)


def _make_sc_body(ch0):
  def _sc_body(l_hbm, idx_hbm, p_hbm, out_hbm,
               buf, idx_all, p_buf, acc, sem_a, sem_b, sem_out):
    cid = lax.axis_index("c")
    sid = lax.axis_index("s")

    pltpu.sync_copy(p_hbm, p_buf)
    pltpu.sync_copy(idx_hbm.at[sid], idx_all)

    z16 = jnp.zeros((16,), jnp.float32)

    def round_body(r, rcarry):
        ch = ch0 + r * 4 + cid
        och = r * NCORE + cid
        pch = p_buf[pl.ds(ch * 16, 16)]

        def in_cp(blk, sem):
            base = sid * EDGES_PER_SUB + blk * EPB
            slot = lax.rem(blk, NSLOT)
            return pltpu.make_async_copy(
                l_hbm.at[pl.ds(base, EPB)],
                buf.at[slot], sem)

        def scat_cp(blk):
            return pltpu.make_async_copy(
                buf.at[lax.rem(blk, NSLOT)],
                acc.at[idx_all.at[blk]], sem_out)

        def zfill(i, carry):
            for t in range(D_FEAT // 16):
                buf[0, i, pl.ds(t * 16, 16)] = z16
            return carry

        lax.fori_loop(0, EPB, zfill, 0)
        zd = [pltpu.async_copy(
                  buf.at[0],
                  acc.at[pl.ds(sid * NODES_PER_SUB + z * EPB, EPB)],
                  sem_a)
              for z in range(ZCOPY)]
        for d in zd:
            d.wait()
        plsc.subcore_barrier()

        in_cp(0, sem_a).start()
        in_cp(1, sem_b).start()

        def do_block(k, sem, first):
            slot = lax.rem(k, NSLOT)
            in_cp(k, sem).wait()

            @plsc.parallel_loop(0, EPB, unroll=4)
            def edge(j):
                for t in range(NCH):
                    lv = buf[slot, j, pl.ds(t * 16, 16)]
                    buf[slot, j, pl.ds(t * 16, 16)] = jnp.exp(pch * lv)

            if first:
                @pl.when(k > 0)
                def _():
                    scat_cp(k - 1).wait()
            else:
                scat_cp(k - 1).wait()

            @pl.when(k + 2 < NBLK)
            def _():
                in_cp(k + 2, sem).start()

            pltpu.async_copy(
                buf.at[slot], acc.at[idx_all.at[k]], sem_out, add=True)

        def pair_body(m, carry):
            do_block(2 * m, sem_a, True)
            do_block(2 * m + 1, sem_b, False)
            return carry

        lax.fori_loop(0, (NBLK - 1) // 2, pair_body, 0)
        do_block(NBLK - 1, sem_a, False)
        scat_cp(NBLK - 1).wait()
        plsc.subcore_barrier()

        pltpu.sync_copy(
            acc.at[pl.ds(sid * NODES_PER_SUB, NODES_PER_SUB)],
            out_hbm.at[pl.ds(och * NPAD + sid * NODES_PER_SUB,
                             NODES_PER_SUB)])
        plsc.subcore_barrier()
        return rcarry

    lax.fori_loop(0, NCH // (2 * NCORE), round_body, 0)
  return _sc_body


def _sc_scatter(l_arr, index, p_arr, ch0):
    mesh = plsc.VectorSubcoreMesh(core_axis_name="c", subcore_axis_name="s")
    f = pl.kernel(
        _make_sc_body(ch0),
        out_type=jax.ShapeDtypeStruct((NCH // 2 * NPAD, D_FEAT), jnp.float32),
        mesh=mesh,
        scratch_types=[
            pltpu.VMEM((NSLOT, EPB, D_FEAT), jnp.float32),
            pltpu.VMEM((NBLK, EPB), jnp.int32),
            pltpu.VMEM((NCH * 16,), jnp.float32),
            pltpu.VMEM_SHARED((NPAD, D_FEAT), jnp.float32),
            pltpu.SemaphoreType.DMA,
            pltpu.SemaphoreType.DMA,
            pltpu.SemaphoreType.DMA,
        ],
    )
    return f(l_arr, index.reshape(NSUB, NBLK, EPB), p_arr)


def _stage3_body(sa_ref, sb_ref, qb_ref, o_ref):
    srcmap = {0: (sa_ref, 0), 1: (sa_ref, 1), 2: (sb_ref, 0), 3: (sb_ref, 1),
              4: (sa_ref, 2), 5: (sa_ref, 3), 6: (sb_ref, 2), 7: (sb_ref, 3)}
    terms = []
    for ch in range(NCH):
        ref, row = srcmap[ch]
        s = jnp.clip(ref[row], EPS, None)
        q = qb_ref[0, ch:ch + 1, :]
        ab = qb_ref[1, ch:ch + 1, :]
        terms.append(jnp.exp(q * jnp.log(s)) * ab)
    for u in range(UNITS):
        num = terms[u] + terms[2 + u]
        den = terms[4 + u] + terms[6 + u]
        mult = 2.0 * jnp.clip(jnp.sign(den), 0.0, None) - 1.0
        den = jnp.where((den < EPS) & (den > -EPS), mult * EPS, den)
        o_ref[u] = num / den


def _stage3(sa_arr, sb_arr, qb_arr):
    return pl.pallas_call(
        _stage3_body,
        grid=(NPAD // NB3,),
        in_specs=[
            pl.BlockSpec((NCH // 2, NB3, D_FEAT), lambda nb: (0, nb, 0)),
            pl.BlockSpec((NCH // 2, NB3, D_FEAT), lambda nb: (0, nb, 0)),
            pl.BlockSpec((2, NCH, 128), lambda nb: (0, 0, 0)),
        ],
        out_specs=pl.BlockSpec((UNITS, NB3, D_FEAT), lambda nb: (0, nb, 0)),
        out_shape=jax.ShapeDtypeStruct((UNITS, NPAD, D_FEAT), jnp.float32),
    )(sa_arr, sb_arr, qb_arr)


def kernel(data, index, W):
    p = jax.nn.relu(W[0:4]).reshape(NCH)
    p_arr = jnp.broadcast_to(p[:, None], (NCH, 16)).reshape(NCH * 16)
    q = jax.nn.relu(W[4:8]).reshape(NCH)
    ab = W[8:12].reshape(NCH)
    qb_arr = jnp.stack([
        jnp.broadcast_to(q[:, None], (NCH, 128)),
        jnp.broadcast_to(ab[:, None], (NCH, 128)),
    ])
    l1_arr = _stage1(data, 0)
    sa_arr = _sc_scatter(l1_arr, index, p_arr, 0)
    l2_arr = _stage1(data, 1)
    sb_arr = _sc_scatter(l2_arr, index, p_arr, 2)
    r_arr = _stage3(sa_arr.reshape(NCH // 2, NPAD, D_FEAT),
                    sb_arr.reshape(NCH // 2, NPAD, D_FEAT), qb_arr)
    return jnp.transpose(r_arr, (1, 2, 0))[:N_NODES]


# speedup vs baseline: 1.5998x; 1.0048x over previous
import jax
import jax.numpy as jnp
from jax import lax
from jax.experimental import pallas as pl
from jax.experimental.pallas import tpu as pltpu
from jax.experimental.pallas import tpu_sc as plsc

N_NODES = 10000
N_EDGES = 160000
D_FEAT = 128
UNITS = 2
EPS = 1e-07

NCH = 8
NSUB = 16
NCORE = 2
EDGES_PER_SUB = N_EDGES // NSUB
EPB = 80
NBLK = EDGES_PER_SUB // EPB
NPAD = 10240
NODES_PER_SUB = NPAD // NSUB
NSLOT = 3
ZCOPY = NODES_PER_SUB // EPB

EB1 = 2000
NB3 = 1280


def _stage1_body(x_ref, o_ref):
    x = jnp.clip(x_ref[...], EPS, 1.0 - EPS)
    o_ref[0] = jnp.log(x)
    o_ref[1] = jnp.log(1.0 - x)


def _stage1(data):
    return pl.pallas_call(
        _stage1_body,
        grid=(N_EDGES // EB1,),
        in_specs=[pl.BlockSpec((EB1, D_FEAT), lambda i: (i, 0))],
        out_specs=pl.BlockSpec((2, EB1, D_FEAT), lambda i: (0, i, 0)),
        out_shape=jax.ShapeDtypeStruct((2, N_EDGES, D_FEAT), jnp.float32),
    )(data)


def _sc_body(l_hbm, idx_hbm, p_hbm, out_hbm,
             buf, idx_all, p_buf, acc, sem_a, sem_b, sem_out):
    cid = lax.axis_index("c")
    sid = lax.axis_index("s")

    pltpu.sync_copy(p_hbm, p_buf)
    pltpu.sync_copy(idx_hbm.at[sid], idx_all)

    z16 = jnp.zeros((16,), jnp.float32)

    def round_body(r, rcarry):
        ch = r * NCORE + cid
        b01 = lax.rem(r, 2)
        pch = p_buf[pl.ds(ch * 16, 16)]

        def in_cp(blk, sem):
            base = sid * EDGES_PER_SUB + blk * EPB
            slot = lax.rem(blk, NSLOT)
            return pltpu.make_async_copy(
                l_hbm.at[pl.ds(b01 * N_EDGES + base, EPB)],
                buf.at[slot], sem)

        def scat_cp(blk):
            return pltpu.make_async_copy(
                buf.at[lax.rem(blk, NSLOT)],
                acc.at[idx_all.at[blk]], sem_out)

        def zfill(i, carry):
            for t in range(D_FEAT // 16):
                buf[0, i, pl.ds(t * 16, 16)] = z16
            return carry

        lax.fori_loop(0, EPB, zfill, 0)
        zd = [pltpu.async_copy(
                  buf.at[0],
                  acc.at[pl.ds(sid * NODES_PER_SUB + z * EPB, EPB)],
                  sem_a)
              for z in range(ZCOPY)]
        for d in zd:
            d.wait()
        plsc.subcore_barrier()

        in_cp(0, sem_a).start()
        in_cp(1, sem_b).start()

        def do_block(k, sem, first):
            slot = lax.rem(k, NSLOT)
            in_cp(k, sem).wait()

            @plsc.parallel_loop(0, EPB, unroll=4)
            def edge(j):
                for t in range(NCH):
                    lv = buf[slot, j, pl.ds(t * 16, 16)]
                    buf[slot, j, pl.ds(t * 16, 16)] = jnp.exp(pch * lv)

            if first:
                @pl.when(k > 0)
                def _():
                    scat_cp(k - 1).wait()
            else:
                scat_cp(k - 1).wait()

            @pl.when(k + 2 < NBLK)
            def _():
                in_cp(k + 2, sem).start()

            pltpu.async_copy(
                buf.at[slot], acc.at[idx_all.at[k]], sem_out, add=True)

        def pair_body(m, carry):
            do_block(2 * m, sem_a, True)
            do_block(2 * m + 1, sem_b, False)
            return carry

        lax.fori_loop(0, (NBLK - 1) // 2, pair_body, 0)
        do_block(NBLK - 1, sem_a, False)
        scat_cp(NBLK - 1).wait()
        plsc.subcore_barrier()

        pltpu.sync_copy(
            acc.at[pl.ds(sid * NODES_PER_SUB, NODES_PER_SUB)],
            out_hbm.at[pl.ds(ch * NPAD + sid * NODES_PER_SUB, NODES_PER_SUB)])
        plsc.subcore_barrier()
        return rcarry

    lax.fori_loop(0, NCH // NCORE, round_body, 0)


def _sc_scatter(l_arr, index, p_arr):
    mesh = plsc.VectorSubcoreMesh(core_axis_name="c", subcore_axis_name="s")
    f = pl.kernel(
        _sc_body,
        out_type=jax.ShapeDtypeStruct((NCH * NPAD, D_FEAT), jnp.float32),
        mesh=mesh,
        scratch_types=[
            pltpu.VMEM((NSLOT, EPB, D_FEAT), jnp.float32),
            pltpu.VMEM((NBLK, EPB), jnp.int32),
            pltpu.VMEM((NCH * 16,), jnp.float32),
            pltpu.VMEM_SHARED((NPAD, D_FEAT), jnp.float32),
            pltpu.SemaphoreType.DMA,
            pltpu.SemaphoreType.DMA,
            pltpu.SemaphoreType.DMA,
        ],
    )
    return f(l_arr.reshape(2 * N_EDGES, D_FEAT),
             index.reshape(NSUB, NBLK, EPB), p_arr)


def _stage3_body(s_ref, qb_ref, o_ref):
    terms = []
    for ch in range(NCH):
        s = jnp.clip(s_ref[ch], EPS, None)
        q = qb_ref[0, ch:ch + 1, :]
        ab = qb_ref[1, ch:ch + 1, :]
        terms.append(jnp.exp(q * jnp.log(s)) * ab)
    for u in range(UNITS):
        num = terms[u] + terms[2 + u]
        den = terms[4 + u] + terms[6 + u]
        mult = 2.0 * jnp.clip(jnp.sign(den), 0.0, None) - 1.0
        den = jnp.where((den < EPS) & (den > -EPS), mult * EPS, den)
        o_ref[u] = num / den


def _stage3(s_arr, qb_arr):
    return pl.pallas_call(
        _stage3_body,
        grid=(NPAD // NB3,),
        in_specs=[
            pl.BlockSpec((NCH, NB3, D_FEAT), lambda nb: (0, nb, 0)),
            pl.BlockSpec((2, NCH, 128), lambda nb: (0, 0, 0)),
        ],
        out_specs=pl.BlockSpec((UNITS, NB3, D_FEAT), lambda nb: (0, nb, 0)),
        out_shape=jax.ShapeDtypeStruct((UNITS, NPAD, D_FEAT), jnp.float32),
    )(s_arr, qb_arr)


def kernel(data, index, W):
    p = jax.nn.relu(W[0:4]).reshape(NCH)
    p_arr = jnp.broadcast_to(p[:, None], (NCH, 16)).reshape(NCH * 16)
    q = jax.nn.relu(W[4:8]).reshape(NCH)
    ab = W[8:12].reshape(NCH)
    qb_arr = jnp.stack([
        jnp.broadcast_to(q[:, None], (NCH, 128)),
        jnp.broadcast_to(ab[:, None], (NCH, 128)),
    ])
    l_arr = _stage1(data)
    s_arr = _sc_scatter(l_arr, index, p_arr).reshape(NCH, NPAD, D_FEAT)
    r_arr = _stage3(s_arr, qb_arr)
    return jnp.transpose(r_arr, (1, 2, 0))[:N_NODES]
